# baseline (speedup 1.0000x reference)
import jax
import jax.numpy as jnp
import numpy as np
from jax.experimental import pallas as pl
from jax.experimental.pallas import tpu as pltpu

_B, _T, _C, _H, _W = 8, 4, 192, 24, 24
_HW = _H * _W
_K = _HW // 2
_CHUNK = 2 * _HW
_MININT = np.int32(-2147483648)


def _threefry2x32(k0, k1, x0, x1):
    rot = [[13, 15, 26, 6], [17, 29, 16, 24]]
    ks = [np.uint32(k0), np.uint32(k1), np.uint32(k0 ^ k1 ^ 0x1BD11BDA)]
    x0 = (x0 + ks[0]).astype(np.uint32)
    x1 = (x1 + ks[1]).astype(np.uint32)

    def rl(v, d):
        return ((v << np.uint32(d)) | (v >> np.uint32(32 - d))).astype(np.uint32)

    for i in range(5):
        for r in rot[i % 2]:
            x0 = (x0 + x1).astype(np.uint32)
            x1 = rl(x1, r) ^ x0
        x0 = (x0 + ks[(i + 1) % 3]).astype(np.uint32)
        x1 = (x1 + ks[(i + 2) % 3] + np.uint32(i + 1)).astype(np.uint32)
    return x0, x1


def _gumbel_const(seed, shape):
    n = int(np.prod(shape))
    x0, x1 = _threefry2x32(np.uint32(seed >> 32), np.uint32(seed & 0xFFFFFFFF),
                           np.zeros(n, np.uint32), np.arange(n, dtype=np.uint32))
    bits = x0 ^ x1
    f = ((bits >> np.uint32(9)) | np.uint32(0x3F800000)).view(np.float32) \
        - np.float32(1.0)
    tiny = np.float32(np.finfo(np.float32).tiny)
    u = np.maximum(tiny, f * (np.float32(1.0) - tiny) + tiny)
    return (-np.log(-np.log(u))).astype(np.float32).reshape(shape)


_GUMBEL = _gumbel_const(42, (_B, _HW))


def _orderable(f):
    b = jax.lax.bitcast_convert_type(f, jnp.int32)
    return jnp.where(b >= 0, b, _MININT - b)


def _mae_body(preds_ref, tgts_ref, xrep_ref, gumb_ref, out_ref, drows, srows):
    g = pl.program_id(0)

    for k in range(2):
        ad = jnp.abs(preds_ref[k] - tgts_ref[k])
        acc = ad[:_HW] + ad[_HW:2 * _HW] + ad[2 * _HW:3 * _HW] + ad[3 * _HW:]
        x = xrep_ref[k]
        norm = jnp.sqrt(jnp.sum(x * x, axis=1, keepdims=True))
        scol = gumb_ref[k] - jnp.log(norm + 1e-7)
        srows[pl.ds(2 * g + k, 1)] = jnp.transpose(scol)
        dcol = jnp.sum(acc, axis=1, keepdims=True)
        drows[pl.ds(2 * g + k, 1)] = jnp.transpose(dcol)

    @pl.when(g == _B // 2 - 1)
    def _select():
        key = _orderable(srows[...])
        u_v = jnp.zeros((_B, 1), jnp.int32)
        for bit in range(31, -1, -1):
            cand_u = jnp.bitwise_or(u_v, jnp.int32(1 << bit) if bit < 31
                                    else _MININT)
            icand = jnp.bitwise_xor(cand_u, _MININT)
            cnt = jnp.sum((key >= icand).astype(jnp.float32), axis=1,
                          keepdims=True)
            u_v = jnp.where(cnt >= float(_K), cand_u, u_v)
        iv = jnp.bitwise_xor(u_v, _MININT)
        gt = key > iv
        eq = key == iv
        n_gt = jnp.sum(gt.astype(jnp.float32), axis=1, keepdims=True)
        need = jnp.float32(_K) - n_gt
        idx = jax.lax.broadcasted_iota(jnp.int32, (_B, _HW), 1)
        m = jnp.zeros((_B, 1), jnp.int32)
        for bit in range(9, -1, -1):
            cand = m + jnp.int32(1 << bit)
            cnt = jnp.sum(jnp.where(eq & (idx < cand), 1.0, 0.0), axis=1,
                          keepdims=True)
            m = jnp.where(cnt < need, cand, m)
        m = m + 1
        mask = gt | (eq & (idx < m))
        out_ref[...] = jnp.sum(jnp.where(mask, drows[...], 0.0),
                               axis=(0, 1), keepdims=True).reshape(1, 1)


def kernel(out_preds, out_targets, tl, tv, x_rep, in_x, in_l, in_v, in_n):
    del tl, tv, in_x, in_l, in_v, in_n
    preds = jnp.transpose(out_preds, (0, 1, 3, 4, 2)).reshape(_B, _T * _HW, _C)
    tgts = jnp.transpose(out_targets, (0, 1, 3, 4, 2)).reshape(_B, _T * _HW, _C)
    xrep = jnp.transpose(x_rep, (0, 2, 3, 1)).reshape(_B, _HW, _C)
    gumb = jnp.asarray(_GUMBEL).reshape(_B, _HW, 1)

    total = pl.pallas_call(
        _mae_body,
        grid=(_B // 2,),
        in_specs=[
            pl.BlockSpec((2, _T * _HW, _C), lambda b: (b, 0, 0)),
            pl.BlockSpec((2, _T * _HW, _C), lambda b: (b, 0, 0)),
            pl.BlockSpec((2, _HW, _C), lambda b: (b, 0, 0)),
            pl.BlockSpec((2, _HW, 1), lambda b: (b, 0, 0)),
        ],
        out_specs=pl.BlockSpec((1, 1), lambda b: (0, 0)),
        out_shape=jax.ShapeDtypeStruct((1, 1), jnp.float32),
        scratch_shapes=[pltpu.VMEM((_B, _HW), jnp.float32),
                        pltpu.VMEM((_B, _HW), jnp.float32)],
    )(preds, tgts, xrep, gumb)
    return total[0, 0] / np.float32(_B * _T * _K * _C)


# speedup vs baseline: 1.0335x; 1.0335x over previous
import jax
import jax.numpy as jnp
import numpy as np
from jax.experimental import pallas as pl
from jax.experimental.pallas import tpu as pltpu

_B, _T, _C, _H, _W = 8, 4, 192, 24, 24
_HW = _H * _W
_K = _HW // 2
_MININT = np.int32(-2147483648)


def _threefry2x32(k0, k1, x0, x1):
    rot = [[13, 15, 26, 6], [17, 29, 16, 24]]
    ks = [np.uint32(k0), np.uint32(k1), np.uint32(k0 ^ k1 ^ 0x1BD11BDA)]
    x0 = (x0 + ks[0]).astype(np.uint32)
    x1 = (x1 + ks[1]).astype(np.uint32)

    def rl(v, d):
        return ((v << np.uint32(d)) | (v >> np.uint32(32 - d))).astype(np.uint32)

    for i in range(5):
        for r in rot[i % 2]:
            x0 = (x0 + x1).astype(np.uint32)
            x1 = rl(x1, r) ^ x0
        x0 = (x0 + ks[(i + 1) % 3]).astype(np.uint32)
        x1 = (x1 + ks[(i + 2) % 3] + np.uint32(i + 1)).astype(np.uint32)
    return x0, x1


def _gumbel_const(seed, shape):
    n = int(np.prod(shape))
    x0, x1 = _threefry2x32(np.uint32(seed >> 32), np.uint32(seed & 0xFFFFFFFF),
                           np.zeros(n, np.uint32), np.arange(n, dtype=np.uint32))
    bits = x0 ^ x1
    f = ((bits >> np.uint32(9)) | np.uint32(0x3F800000)).view(np.float32) \
        - np.float32(1.0)
    tiny = np.float32(np.finfo(np.float32).tiny)
    u = np.maximum(tiny, f * (np.float32(1.0) - tiny) + tiny)
    return (-np.log(-np.log(u))).astype(np.float32).reshape(shape)


_GUMBEL = _gumbel_const(42, (_B, _HW))


def _orderable(f):
    b = jax.lax.bitcast_convert_type(f, jnp.int32)
    return jnp.where(b >= 0, b, _MININT - b)


def _mae_body(preds_ref, tgts_ref, xrep_ref, gumb_ref, out_ref, drows, srows):
    b = pl.program_id(0)

    ad = jnp.abs(preds_ref[0] - tgts_ref[0])
    acc = ad[:_HW] + ad[_HW:2 * _HW] + ad[2 * _HW:3 * _HW] + ad[3 * _HW:]

    x = xrep_ref[0]
    norm = jnp.sqrt(jnp.sum(x * x, axis=1, keepdims=True))
    scol = gumb_ref[0] - jnp.log(norm + 1e-7)
    srows[pl.ds(b, 1)] = jnp.transpose(scol)
    dcol = jnp.sum(acc, axis=1, keepdims=True)
    drows[pl.ds(b, 1)] = jnp.transpose(dcol)

    @pl.when(b == _B - 1)
    def _select():
        key = _orderable(srows[...])
        u_v = jnp.zeros((_B, 1), jnp.int32)
        for bit in range(31, -1, -1):
            cand_u = jnp.bitwise_or(u_v, jnp.int32(1 << bit) if bit < 31
                                    else _MININT)
            icand = jnp.bitwise_xor(cand_u, _MININT)
            cnt = jnp.sum((key >= icand).astype(jnp.float32), axis=1,
                          keepdims=True)
            u_v = jnp.where(cnt >= float(_K), cand_u, u_v)
        iv = jnp.bitwise_xor(u_v, _MININT)
        gt = key > iv
        eq = key == iv
        n_gt = jnp.sum(gt.astype(jnp.float32), axis=1, keepdims=True)
        need = jnp.float32(_K) - n_gt
        idx = jax.lax.broadcasted_iota(jnp.int32, (_B, _HW), 1)
        m = jnp.zeros((_B, 1), jnp.int32)
        for bit in range(9, -1, -1):
            cand = m + jnp.int32(1 << bit)
            cnt = jnp.sum(jnp.where(eq & (idx < cand), 1.0, 0.0), axis=1,
                          keepdims=True)
            m = jnp.where(cnt < need, cand, m)
        m = m + 1
        mask = gt | (eq & (idx < m))
        out_ref[...] = jnp.sum(jnp.where(mask, drows[...], 0.0),
                               axis=(0, 1), keepdims=True).reshape(1, 1)


def kernel(out_preds, out_targets, tl, tv, x_rep, in_x, in_l, in_v, in_n):
    del tl, tv, in_x, in_l, in_v, in_n
    preds = jnp.transpose(out_preds, (0, 1, 3, 4, 2)).reshape(_B, _T * _HW, _C)
    tgts = jnp.transpose(out_targets, (0, 1, 3, 4, 2)).reshape(_B, _T * _HW, _C)
    xrep = jnp.transpose(x_rep, (0, 2, 3, 1)).reshape(_B, _HW, _C)
    gumb = jnp.asarray(_GUMBEL).reshape(_B, _HW, 1)

    total = pl.pallas_call(
        _mae_body,
        grid=(_B,),
        in_specs=[
            pl.BlockSpec((1, _T * _HW, _C), lambda b: (b, 0, 0)),
            pl.BlockSpec((1, _T * _HW, _C), lambda b: (b, 0, 0)),
            pl.BlockSpec((1, _HW, _C), lambda b: (b, 0, 0)),
            pl.BlockSpec((1, _HW, 1), lambda b: (b, 0, 0)),
        ],
        out_specs=pl.BlockSpec((1, 1), lambda b: (0, 0)),
        out_shape=jax.ShapeDtypeStruct((1, 1), jnp.float32),
        scratch_shapes=[pltpu.VMEM((_B, _HW), jnp.float32),
                        pltpu.VMEM((_B, _HW), jnp.float32)],
    )(preds, tgts, xrep, gumb)
    return total[0, 0] / np.float32(_B * _T * _K * _C)
